# baseline (speedup 1.0000x reference)
import jax
import jax.numpy as jnp
from jax.experimental import pallas as pl
from jax.experimental.pallas import tpu as pltpu

B, CIN, L = 64, 128, 2048
C0, C1, C2 = 32, 64, 128
E, D, NC = 8, 192, 1000
NTOT = float(B * L)
EPS = 1e-5


def _affine(s_ref, q_ref, g_ref, b_ref):
    mean = s_ref[...] / NTOT
    var = q_ref[...] / NTOT - mean * mean
    scale = g_ref[...] * jax.lax.rsqrt(var + EPS)
    shift = b_ref[...] - mean * scale
    return scale, shift


def _conv(h, w_ref, taps, cout):
    pad = (taps - 1) // 2
    hp = jnp.pad(h, ((pad, pad), (0, 0)))
    acc = jnp.zeros((L, cout), jnp.float32)
    for k in range(taps):
        acc += jnp.dot(hp[k:k + L], w_ref[k], preferred_element_type=jnp.float32)
    return acc


def _rowsum(a):
    ones = jnp.ones((8, a.shape[0]), jnp.float32)
    return jnp.dot(ones, a, preferred_element_type=jnp.float32)[0:1]


def _pack(a, g):
    n = a.shape[0] // g
    return jnp.concatenate([a[i * n:(i + 1) * n] for i in range(g)], axis=1)


def _unpack(a, g):
    c = a.shape[1] // g
    return jnp.concatenate([a[:, i * c:(i + 1) * c] for i in range(g)], axis=0)


def _backbone_kern(x_ref, w0_ref, w1_ref, w2_ref,
                   g0_ref, b0_ref, g1_ref, b1_ref, g2_ref, b2_ref,
                   p_ref,
                   y0_scr, y1_scr, s0, q0, s1, q1, s2, q2):
    p = pl.program_id(0)
    b = pl.program_id(1)

    @pl.when((p == 0) & (b == 0))
    def _init():
        for r in (s0, q0, s1, q1, s2, q2):
            r[...] = jnp.zeros_like(r)

    @pl.when(p == 0)
    def _phase0():
        xt = x_ref[0].T
        acc = _conv(xt, w0_ref, 5, C0)
        y0_scr[b] = _pack(acc, 4)
        s0[...] += _rowsum(acc)
        q0[...] += _rowsum(acc * acc)

    @pl.when(p == 1)
    def _phase1():
        scale, shift = _affine(s0, q0, g0_ref, b0_ref)
        h = jnp.maximum(_unpack(y0_scr[b], 4) * scale + shift, 0.0)
        acc = _conv(h, w1_ref, 3, C1)
        y1_scr[b] = _pack(acc, 2)
        s1[...] += _rowsum(acc)
        q1[...] += _rowsum(acc * acc)

    @pl.when(p == 2)
    def _phase2():
        scale, shift = _affine(s1, q1, g1_ref, b1_ref)
        h = jnp.maximum(_unpack(y1_scr[b], 2) * scale + shift, 0.0)
        acc = _conv(h, w2_ref, 3, C2)
        s2[...] += _rowsum(acc)
        q2[...] += _rowsum(acc * acc)

    @pl.when(p == 3)
    def _phase3():
        scale, shift = _affine(s1, q1, g1_ref, b1_ref)
        h = jnp.maximum(_unpack(y1_scr[b], 2) * scale + shift, 0.0)
        acc = _conv(h, w2_ref, 3, C2)
        scale2, shift2 = _affine(s2, q2, g2_ref, b2_ref)
        h2 = jnp.maximum(acc * scale2 + shift2, 0.0)
        p_ref[0] = _rowsum(h2) * (1.0 / L)


def _head_kern(p_ref, fcw_ref, fcb_ref, gw_ref, gb_ref, ew_ref, eb_ref,
               out_ref, loss_ref):
    pooled = p_ref[...]
    out = jnp.maximum(
        jnp.dot(pooled, fcw_ref[...], preferred_element_type=jnp.float32)
        + fcb_ref[...], 0.0)
    logits = jnp.dot(out, gw_ref[...], preferred_element_type=jnp.float32) + gb_ref[...]
    mx = jnp.max(logits, axis=1, keepdims=True)
    ex = jnp.exp(logits - mx)
    gw = ex / jnp.sum(ex, axis=1, keepdims=True)

    iota = jax.lax.broadcasted_iota(jnp.int32, (B, E), 1)
    m1 = jnp.max(gw, axis=1, keepdims=True)
    i1 = jnp.min(jnp.where(gw >= m1, iota, E), axis=1, keepdims=True)
    g2 = jnp.where(iota == i1, -1.0, gw)
    m2 = jnp.max(g2, axis=1, keepdims=True)
    i2 = jnp.min(jnp.where(g2 >= m2, iota, E), axis=1, keepdims=True)
    cw = jnp.where(iota == i1, m1, 0.0) + jnp.where(iota == i2, m2, 0.0)

    acc = jnp.zeros((B, NC), jnp.float32)
    for e in range(E):
        se = jnp.dot(out, ew_ref[e], preferred_element_type=jnp.float32) \
            + eb_ref[e:e + 1, :]
        acc += cw[:, e:e + 1] * se
    out_ref[...] = acc

    mu = jnp.sum(gw, axis=0, keepdims=True) * (1.0 / B)
    entropy = -jnp.sum(mu * jnp.log(mu + 1e-10))
    variance = jnp.sum((mu - 1.0 / E) ** 2)
    loss_ref[...] = jnp.full((1, 1), variance + 0.1 * entropy, jnp.float32)


@jax.jit
def kernel(x, w0, g0, b0, w1, g1, b1, w2, g2, b2, fcW, fcB, gateW, gateB,
           expW, expB):
    f32 = jnp.float32
    w0t = jnp.transpose(w0, (2, 1, 0))
    w1t = jnp.transpose(w1, (2, 1, 0))
    w2t = jnp.transpose(w2, (2, 1, 0))
    ewt = jnp.transpose(expW, (0, 2, 1))

    def cspec(c):
        return pl.BlockSpec((1, c), lambda p, b: (0, 0))

    def wspec(t, ci, co):
        return pl.BlockSpec((t, ci, co), lambda p, b: (0, 0, 0))

    pooled = pl.pallas_call(
        _backbone_kern,
        grid=(4, B),
        in_specs=[
            pl.BlockSpec((1, CIN, L),
                         lambda p, b: (jnp.where(p == 0, b, 0), 0, 0)),
            wspec(5, CIN, C0), wspec(3, C0, C1), wspec(3, C1, C2),
            cspec(C0), cspec(C0), cspec(C1), cspec(C1), cspec(C2), cspec(C2),
        ],
        out_specs=pl.BlockSpec((1, 1, C2),
                               lambda p, b: (jnp.where(p == 3, b, 0), 0, 0)),
        out_shape=jax.ShapeDtypeStruct((B, 1, C2), f32),
        scratch_shapes=[
            pltpu.VMEM((B, L // 4, C0 * 4), f32),
            pltpu.VMEM((B, L // 2, C1 * 2), f32),
            pltpu.VMEM((1, C0), f32), pltpu.VMEM((1, C0), f32),
            pltpu.VMEM((1, C1), f32), pltpu.VMEM((1, C1), f32),
            pltpu.VMEM((1, C2), f32), pltpu.VMEM((1, C2), f32),
        ],
    )(x, w0t, w1t, w2t,
      g0.reshape(1, C0), b0.reshape(1, C0),
      g1.reshape(1, C1), b1.reshape(1, C1),
      g2.reshape(1, C2), b2.reshape(1, C2))
    pooled = pooled.reshape(B, C2)

    final, loss = pl.pallas_call(
        _head_kern,
        out_shape=[
            jax.ShapeDtypeStruct((B, NC), f32),
            jax.ShapeDtypeStruct((1, 1), f32),
        ],
    )(pooled, fcW.T, fcB.reshape(1, D), gateW.T, gateB.reshape(1, E),
      ewt, expB)

    return final, loss[0, 0]


# speedup vs baseline: 1.3280x; 1.3280x over previous
import jax
import jax.numpy as jnp
from jax.experimental import pallas as pl
from jax.experimental.pallas import tpu as pltpu

B, CIN, L = 64, 128, 2048
C0, C1, C2 = 32, 64, 128
E, D, NC = 8, 192, 1000
NTOT = float(B * L)
EPS = 1e-5
GB = 2


def _affine(s_ref, q_ref, g_ref, b_ref):
    mean = s_ref[...] / NTOT
    var = q_ref[...] / NTOT - mean * mean
    scale = g_ref[...] * jax.lax.rsqrt(var + EPS)
    shift = b_ref[...] - mean * scale
    return scale, shift


def _conv(h, w_ref, taps, cout):
    pad = (taps - 1) // 2
    hp = jnp.pad(h, ((pad, pad), (0, 0)))
    acc = jnp.zeros((L, cout), jnp.float32)
    for k in range(taps):
        acc += jnp.dot(hp[k:k + L], w_ref[k], preferred_element_type=jnp.float32)
    return acc


def _pack(a, g):
    n = a.shape[0] // g
    return jnp.concatenate([a[i * n:(i + 1) * n] for i in range(g)], axis=1)


def _unpack(a, g):
    c = a.shape[1] // g
    return jnp.concatenate([a[:, i * c:(i + 1) * c] for i in range(g)], axis=0)


def _backbone_kern(x_ref, w0_ref, w1_ref, w2_ref,
                   g0_ref, b0_ref, g1_ref, b1_ref, g2_ref, b2_ref,
                   p_ref,
                   y0_scr, y1_scr, s0, q0, s1, q1, s2, q2):
    p = pl.program_id(0)
    j = pl.program_id(1)

    @pl.when((p == 0) & (j == 0))
    def _init():
        for r in (s0, q0, s1, q1, s2, q2):
            r[...] = jnp.zeros_like(r)

    @pl.when(p == 0)
    def _phase0():
        ps = jnp.zeros((1, C0), jnp.float32)
        pq = jnp.zeros((1, C0), jnp.float32)
        for s in range(GB):
            xt = x_ref[s].T
            acc = _conv(xt, w0_ref, 5, C0)
            y0_scr[j * GB + s] = _pack(acc, 4)
            ps += jnp.sum(acc, axis=0, keepdims=True)
            pq += jnp.sum(acc * acc, axis=0, keepdims=True)
        s0[...] += ps
        q0[...] += pq

    @pl.when(p == 1)
    def _phase1():
        scale, shift = _affine(s0, q0, g0_ref, b0_ref)
        ps = jnp.zeros((1, C1), jnp.float32)
        pq = jnp.zeros((1, C1), jnp.float32)
        for s in range(GB):
            h = jnp.maximum(_unpack(y0_scr[j * GB + s], 4) * scale + shift, 0.0)
            acc = _conv(h, w1_ref, 3, C1)
            y1_scr[j * GB + s] = _pack(acc, 2)
            ps += jnp.sum(acc, axis=0, keepdims=True)
            pq += jnp.sum(acc * acc, axis=0, keepdims=True)
        s1[...] += ps
        q1[...] += pq

    @pl.when(p == 2)
    def _phase2():
        scale, shift = _affine(s1, q1, g1_ref, b1_ref)
        ps = jnp.zeros((1, C2), jnp.float32)
        pq = jnp.zeros((1, C2), jnp.float32)
        for s in range(GB):
            h = jnp.maximum(_unpack(y1_scr[j * GB + s], 2) * scale + shift, 0.0)
            acc = _conv(h, w2_ref, 3, C2)
            ps += jnp.sum(acc, axis=0, keepdims=True)
            pq += jnp.sum(acc * acc, axis=0, keepdims=True)
        s2[...] += ps
        q2[...] += pq

    @pl.when(p == 3)
    def _phase3():
        scale, shift = _affine(s1, q1, g1_ref, b1_ref)
        scale2, shift2 = _affine(s2, q2, g2_ref, b2_ref)
        rows = []
        for s in range(GB):
            h = jnp.maximum(_unpack(y1_scr[j * GB + s], 2) * scale + shift, 0.0)
            acc = _conv(h, w2_ref, 3, C2)
            h2 = jnp.maximum(acc * scale2 + shift2, 0.0)
            rows.append(jnp.sum(h2, axis=0, keepdims=True) * (1.0 / L))
        p_ref[0] = jnp.concatenate(rows, axis=0)


def _head_kern(p_ref, fcw_ref, fcb_ref, gw_ref, gb_ref, ew_ref, eb_ref,
               out_ref, loss_ref):
    pooled = p_ref[...]
    out = jnp.maximum(
        jnp.dot(pooled, fcw_ref[...], preferred_element_type=jnp.float32)
        + fcb_ref[...], 0.0)
    logits = jnp.dot(out, gw_ref[...], preferred_element_type=jnp.float32) + gb_ref[...]
    mx = jnp.max(logits, axis=1, keepdims=True)
    ex = jnp.exp(logits - mx)
    gw = ex / jnp.sum(ex, axis=1, keepdims=True)

    iota = jax.lax.broadcasted_iota(jnp.int32, (B, E), 1)
    m1 = jnp.max(gw, axis=1, keepdims=True)
    i1 = jnp.min(jnp.where(gw >= m1, iota, E), axis=1, keepdims=True)
    g2 = jnp.where(iota == i1, -1.0, gw)
    m2 = jnp.max(g2, axis=1, keepdims=True)
    i2 = jnp.min(jnp.where(g2 >= m2, iota, E), axis=1, keepdims=True)
    cw = jnp.where(iota == i1, m1, 0.0) + jnp.where(iota == i2, m2, 0.0)

    acc = jnp.zeros((B, NC), jnp.float32)
    for e in range(E):
        se = jnp.dot(out, ew_ref[e], preferred_element_type=jnp.float32) \
            + eb_ref[e:e + 1, :]
        acc += cw[:, e:e + 1] * se
    out_ref[...] = acc

    mu = jnp.sum(gw, axis=0, keepdims=True) * (1.0 / B)
    entropy = -jnp.sum(mu * jnp.log(mu + 1e-10))
    variance = jnp.sum((mu - 1.0 / E) ** 2)
    loss_ref[...] = jnp.full((1, 1), variance + 0.1 * entropy, jnp.float32)


@jax.jit
def kernel(x, w0, g0, b0, w1, g1, b1, w2, g2, b2, fcW, fcB, gateW, gateB,
           expW, expB):
    f32 = jnp.float32
    w0t = jnp.transpose(w0, (2, 1, 0))
    w1t = jnp.transpose(w1, (2, 1, 0))
    w2t = jnp.transpose(w2, (2, 1, 0))
    ewt = jnp.transpose(expW, (0, 2, 1))

    def cspec(c):
        return pl.BlockSpec((1, c), lambda p, b: (0, 0))

    def wspec(t, ci, co):
        return pl.BlockSpec((t, ci, co), lambda p, b: (0, 0, 0))

    pooled = pl.pallas_call(
        _backbone_kern,
        grid=(4, B // GB),
        in_specs=[
            pl.BlockSpec((GB, CIN, L),
                         lambda p, b: (jnp.where(p == 0, b, 0), 0, 0)),
            wspec(5, CIN, C0), wspec(3, C0, C1), wspec(3, C1, C2),
            cspec(C0), cspec(C0), cspec(C1), cspec(C1), cspec(C2), cspec(C2),
        ],
        out_specs=pl.BlockSpec((1, GB, C2),
                               lambda p, b: (jnp.where(p == 3, b, 0), 0, 0)),
        out_shape=jax.ShapeDtypeStruct((B // GB, GB, C2), f32),
        scratch_shapes=[
            pltpu.VMEM((B, L // 4, C0 * 4), f32),
            pltpu.VMEM((B, L // 2, C1 * 2), f32),
            pltpu.VMEM((1, C0), f32), pltpu.VMEM((1, C0), f32),
            pltpu.VMEM((1, C1), f32), pltpu.VMEM((1, C1), f32),
            pltpu.VMEM((1, C2), f32), pltpu.VMEM((1, C2), f32),
        ],
    )(x, w0t, w1t, w2t,
      g0.reshape(1, C0), b0.reshape(1, C0),
      g1.reshape(1, C1), b1.reshape(1, C1),
      g2.reshape(1, C2), b2.reshape(1, C2))
    pooled = pooled.reshape(B, C2)

    final, loss = pl.pallas_call(
        _head_kern,
        out_shape=[
            jax.ShapeDtypeStruct((B, NC), f32),
            jax.ShapeDtypeStruct((1, 1), f32),
        ],
    )(pooled, fcW.T, fcB.reshape(1, D), gateW.T, gateB.reshape(1, E),
      ewt, expB)

    return final, loss[0, 0]
